# baseline (speedup 1.0000x reference)
import functools

import jax
import jax.numpy as jnp
from jax import lax
from jax.experimental import pallas as pl
from jax.experimental.pallas import tpu as pltpu
from jax.experimental.pallas import tpu_sc as plsc

B = 16384
D = 128
NC = 2
NS = 16
NW = NC * NS
BPW = B // NW
CH = 128
NCH = BPW // CH
NCHUNK16 = D // 16


def _tree_sum(vals):
    vals = list(vals)
    while len(vals) > 1:
        nxt = [vals[i] + vals[i + 1] for i in range(0, len(vals) - 1, 2)]
        if len(vals) % 2:
            nxt.append(vals[-1])
        vals = nxt
    return vals[0]


_mesh = plsc.VectorSubcoreMesh(core_axis_name="c", subcore_axis_name="s")


@functools.partial(
    pl.kernel,
    mesh=_mesh,
    compiler_params=pltpu.CompilerParams(needs_layout_passes=False),
    out_type=jax.ShapeDtypeStruct((B,), jnp.float32),
    scratch_types=[
        pltpu.VMEM((BPW,), jnp.int32),
        pltpu.VMEM((BPW,), jnp.int32),
        pltpu.VMEM((CH, D), jnp.float32),
        pltpu.VMEM((CH, D), jnp.float32),
        pltpu.VMEM((CH, D), jnp.float32),
        pltpu.VMEM((CH, D), jnp.float32),
        pltpu.VMEM((2 * D + 16,), jnp.float32),
        pltpu.VMEM((8 * 256,), jnp.float32),
        pltpu.VMEM((BPW,), jnp.float32),
        pltpu.SemaphoreType.DMA,
        pltpu.SemaphoreType.DMA,
    ],
)
def _sc_recommender(uidx_hbm, midx_hbm, utab_hbm, mtab_hbm, wb_hbm,
                    out_hbm,
                    uidx_v, midx_v, ub0, ub1, mb0, mb1, wv, tr, ob,
                    sem0, sem1):
    wid = lax.axis_index("s") * NC + lax.axis_index("c")
    base = wid * BPW

    with jax.named_scope("stage_idx"):
        cu_i = pltpu.async_copy(uidx_hbm.at[pl.ds(base, BPW)], uidx_v, sem0)
        cm_i = pltpu.async_copy(midx_hbm.at[pl.ds(base, BPW)], midx_v, sem1)
        pltpu.sync_copy(wb_hbm, wv)
        cu_i.wait()
        cm_i.wait()

    ubufs = (ub0, ub1)
    mbufs = (mb0, mb1)
    sems = (sem0, sem1)

    def start(ci, bi):
        iu = uidx_v.at[pl.ds(ci * CH, CH)]
        im = midx_v.at[pl.ds(ci * CH, CH)]
        cu = pltpu.async_copy(utab_hbm.at[iu], ubufs[bi], sems[bi])
        cm = pltpu.async_copy(mtab_hbm.at[im], mbufs[bi], sems[bi])
        return cu, cm

    inflight = [None, None]
    inflight[0] = start(0, 0)

    wu = [wv[pl.ds(c * 16, 16)] for c in range(NCHUNK16)]
    wm = [wv[pl.ds(D + c * 16, 16)] for c in range(NCHUNK16)]
    bias = wv[pl.ds(2 * D, 16)]
    lane = lax.iota(jnp.int32, 16)
    base16 = lane * 16

    for ci in range(NCH):
        bi = ci % 2
        if ci + 1 < NCH:
            inflight[1 - bi] = start(ci + 1, 1 - bi)
        cu, cm = inflight[bi]
        with jax.named_scope(f"wait{ci}"):
            cu.wait()
            cm.wait()
        ubv = ubufs[bi]
        mbv = mbufs[bi]

        ns = jax.named_scope(f"compute{ci}")
        ns.__enter__()

        @plsc.parallel_loop(0, CH // 16)
        def group(g, ubv=ubv, mbv=mbv, ci=ci):
            tb = g * 256
            for r in range(16):
                row = g * 16 + r
                prods = []
                for c in range(NCHUNK16):
                    prods.append(ubv[row, pl.ds(c * 16, 16)] * wu[c])
                    prods.append(mbv[row, pl.ds(c * 16, 16)] * wm[c])
                tr[pl.ds(tb + r * 16, 16)] = _tree_sum(prods)
            cols = [
                plsc.load_gather(tr, [tb + base16 + ((lane + j) & 15)])
                for j in range(16)
            ]
            acc = _tree_sum(cols) + bias
            y = 1.0 / (1.0 + jnp.exp(-acc))
            ob[pl.ds(ci * CH + g * 16, 16)] = y

        ns.__exit__(None, None, None)

    with jax.named_scope("writeback"):
        pltpu.sync_copy(ob, out_hbm.at[pl.ds(base, BPW)])


def kernel(user_indices, movie_indices, user_table, movie_table, fc_w, fc_b):
    wb = jnp.concatenate([
        fc_w.reshape(2 * D).astype(jnp.float32),
        jnp.broadcast_to(fc_b.reshape(()), (16,)).astype(jnp.float32),
    ])
    out = _sc_recommender(
        user_indices.astype(jnp.int32),
        movie_indices.astype(jnp.int32),
        user_table,
        movie_table,
        wb,
    )
    return out.reshape(B, 1)


# speedup vs baseline: 1.0449x; 1.0449x over previous
import functools

import jax
import jax.numpy as jnp
from jax import lax
from jax.experimental import pallas as pl
from jax.experimental.pallas import tpu as pltpu
from jax.experimental.pallas import tpu_sc as plsc

B = 16384
D = 128
NC = 2
NS = 16
NW = NC * NS
BPW = B // NW
CH = 128
NCH = BPW // CH
NCHUNK16 = D // 16


def _tree_sum(vals):
    vals = list(vals)
    while len(vals) > 1:
        nxt = [vals[i] + vals[i + 1] for i in range(0, len(vals) - 1, 2)]
        if len(vals) % 2:
            nxt.append(vals[-1])
        vals = nxt
    return vals[0]


_mesh = plsc.VectorSubcoreMesh(core_axis_name="c", subcore_axis_name="s")


@functools.partial(
    pl.kernel,
    mesh=_mesh,
    compiler_params=pltpu.CompilerParams(needs_layout_passes=False),
    out_type=jax.ShapeDtypeStruct((B,), jnp.float32),
    scratch_types=[
        pltpu.VMEM((BPW,), jnp.int32),
        pltpu.VMEM((BPW,), jnp.int32),
        pltpu.VMEM((CH, D), jnp.float32),
        pltpu.VMEM((CH, D), jnp.float32),
        pltpu.VMEM((CH, D), jnp.float32),
        pltpu.VMEM((CH, D), jnp.float32),
        pltpu.VMEM((2 * D + 16,), jnp.float32),
        pltpu.VMEM((8 * 256,), jnp.float32),
        pltpu.VMEM((BPW,), jnp.float32),
        pltpu.SemaphoreType.DMA,
        pltpu.SemaphoreType.DMA,
    ],
)
def _sc_recommender(uidx_hbm, midx_hbm, utab_hbm, mtab_hbm, wb_hbm,
                    out_hbm,
                    uidx_v, midx_v, ub0, ub1, mb0, mb1, wv, tr, ob,
                    sem0, sem1):
    wid = lax.axis_index("s") * NC + lax.axis_index("c")
    base = wid * BPW

    with jax.named_scope("stage_idx"):
        cu_i = pltpu.async_copy(uidx_hbm.at[pl.ds(base, BPW)], uidx_v, sem0)
        cm_i = pltpu.async_copy(midx_hbm.at[pl.ds(base, BPW)], midx_v, sem1)
        cu_i.wait()
        cm_i.wait()

    ubufs = (ub0, ub1)
    mbufs = (mb0, mb1)
    sems = (sem0, sem1)

    def start(ci, bi):
        iu = uidx_v.at[pl.ds(ci * CH, CH)]
        im = midx_v.at[pl.ds(ci * CH, CH)]
        cu = pltpu.async_copy(utab_hbm.at[iu], ubufs[bi], sems[bi])
        cm = pltpu.async_copy(mtab_hbm.at[im], mbufs[bi], sems[bi])
        return cu, cm

    inflight = [None, None]
    inflight[0] = start(0, 0)
    pltpu.sync_copy(wb_hbm, wv)

    wu = [wv[pl.ds(c * 16, 16)] for c in range(NCHUNK16)]
    wm = [wv[pl.ds(D + c * 16, 16)] for c in range(NCHUNK16)]
    bias = wv[pl.ds(2 * D, 16)]
    lane = lax.iota(jnp.int32, 16)
    base16 = lane * 16

    for ci in range(NCH):
        bi = ci % 2
        if ci + 1 < NCH:
            inflight[1 - bi] = start(ci + 1, 1 - bi)
        cu, cm = inflight[bi]
        with jax.named_scope(f"wait{ci}"):
            cu.wait()
            cm.wait()
        ubv = ubufs[bi]
        mbv = mbufs[bi]

        ns = jax.named_scope(f"compute{ci}")
        ns.__enter__()

        @plsc.parallel_loop(0, CH // 16)
        def group(g, ubv=ubv, mbv=mbv, ci=ci):
            tb = g * 256
            for r in range(16):
                row = g * 16 + r
                prods = []
                for c in range(NCHUNK16):
                    prods.append(ubv[row, pl.ds(c * 16, 16)] * wu[c])
                    prods.append(mbv[row, pl.ds(c * 16, 16)] * wm[c])
                tr[pl.ds(tb + r * 16, 16)] = _tree_sum(prods)
            cols = [
                plsc.load_gather(tr, [tb + base16 + ((lane + j) & 15)])
                for j in range(16)
            ]
            acc = _tree_sum(cols) + bias
            y = 1.0 / (1.0 + jnp.exp(-acc))
            ob[pl.ds(ci * CH + g * 16, 16)] = y

        ns.__exit__(None, None, None)

    with jax.named_scope("writeback"):
        pltpu.sync_copy(ob, out_hbm.at[pl.ds(base, BPW)])


def kernel(user_indices, movie_indices, user_table, movie_table, fc_w, fc_b):
    wb = jnp.concatenate([
        fc_w.reshape(2 * D).astype(jnp.float32),
        jnp.broadcast_to(fc_b.reshape(()), (16,)).astype(jnp.float32),
    ])
    out = _sc_recommender(
        user_indices.astype(jnp.int32),
        movie_indices.astype(jnp.int32),
        user_table,
        movie_table,
        wb,
    )
    return out.reshape(B, 1)
